# baseline (speedup 1.0000x reference)
import jax
import jax.numpy as jnp
from jax import lax
from jax.experimental import pallas as pl
from jax.experimental.pallas import tpu as pltpu
from jax.experimental.pallas import tpu_sc as plsc

NC = 2
NS = 16
NW = NC * NS

VOCAB = 100000
MAXLEN = 200
D = 32
BATCH = 4096
GROUPS = BATCH // 16
NPAIR = D // 2
THALF = MAXLEN // 2
NBUF = 2

MASK_HI = jnp.int32(-65536)


def _body(xt_hbm, tokP_hbm, pos_hbm, out_hbm, row_v, idx0, idx1,
          bufa0, bufb0, bufa1, bufb1, pos_v, si0, si1, ss0, ss1):
    wid = lax.axis_index("s") * NC + lax.axis_index("c")
    p = wid % NPAIR
    t0 = (wid // NPAIR) * THALF
    d0 = 2 * p
    d1 = 2 * p + 1

    idx = (idx0, idx1)
    bufa = (bufa0, bufa1)
    bufb = (bufb0, bufb1)
    sem_i = (si0, si1)
    sem_s = (ss0, ss1)

    pltpu.sync_copy(tokP_hbm.at[p], row_v)
    pltpu.sync_copy(pos_hbm, pos_v)

    zeros = lax.iota(jnp.int32, 16) * 0

    def step(k, r, drain, refire):
        t = t0 + k
        pltpu.make_async_copy(xt_hbm.at[t], idx[r], sem_i[r]).wait()

        pb0 = plsc.load_gather(pos_v, [zeros + t, zeros + d0])
        pb1 = plsc.load_gather(pos_v, [zeros + t, zeros + d1])

        if drain:
            pltpu.make_async_copy(bufa[r], out_hbm.at[t - NBUF, d0],
                                  sem_s[r]).wait()
            pltpu.make_async_copy(bufb[r], out_hbm.at[t - NBUF, d1],
                                  sem_s[r]).wait()

        src = idx[r]
        da = bufa[r]
        db = bufb[r]

        @plsc.parallel_loop(0, GROUPS, unroll=16)
        def _(g):
            iv = src[pl.ds(g * 16, 16)]
            w = plsc.load_gather(row_v, [iv])
            lo = plsc.bitcast(lax.shift_left(w, 16), jnp.float32)
            hi = plsc.bitcast(lax.bitwise_and(w, MASK_HI), jnp.float32)
            da[pl.ds(g * 16, 16)] = lo + pb0
            db[pl.ds(g * 16, 16)] = hi + pb1

        pltpu.async_copy(bufa[r], out_hbm.at[t, d0], sem_s[r])
        pltpu.async_copy(bufb[r], out_hbm.at[t, d1], sem_s[r])

        if refire:
            pltpu.async_copy(xt_hbm.at[t + NBUF], idx[r], sem_i[r])

    for r in range(NBUF):
        pltpu.async_copy(xt_hbm.at[t0 + r], idx[r], sem_i[r])

    step(0, 0, drain=False, refire=True)
    step(1, 1, drain=False, refire=True)

    def ring_body(j, _):
        for r in range(NBUF):
            step(NBUF * j + r, r, drain=True, refire=True)
        return 0

    lax.fori_loop(1, 49, ring_body, 0)

    step(THALF - 2, 0, drain=True, refire=False)
    step(THALF - 1, 1, drain=True, refire=False)

    for r in range(NBUF):
        k = THALF - 2 + r
        pltpu.make_async_copy(bufa[r], out_hbm.at[t0 + k, d0],
                              sem_s[r]).wait()
        pltpu.make_async_copy(bufb[r], out_hbm.at[t0 + k, d1],
                              sem_s[r]).wait()


@jax.jit
def _embed(xt, tokT, pos_table):
    u = lax.bitcast_convert_type(tokT.astype(jnp.bfloat16),
                                 jnp.uint16).astype(jnp.uint32)
    u3 = u.reshape(NPAIR, 2, VOCAB)
    shifts = jnp.array([0, 16], dtype=jnp.uint32)[None, :, None]
    packed = jnp.sum(jnp.left_shift(u3, shifts), axis=1)
    tokP = lax.bitcast_convert_type(packed, jnp.int32)
    mesh = plsc.VectorSubcoreMesh(core_axis_name="c", subcore_axis_name="s")
    return pl.kernel(
        _body,
        out_type=jax.ShapeDtypeStruct((MAXLEN, D, BATCH), jnp.float32),
        mesh=mesh,
        scratch_types=[
            pltpu.VMEM((VOCAB,), jnp.int32),
            pltpu.VMEM((BATCH,), jnp.int32),
            pltpu.VMEM((BATCH,), jnp.int32),
            pltpu.VMEM((BATCH,), jnp.float32),
            pltpu.VMEM((BATCH,), jnp.float32),
            pltpu.VMEM((BATCH,), jnp.float32),
            pltpu.VMEM((BATCH,), jnp.float32),
            pltpu.VMEM((MAXLEN, D), jnp.float32),
            pltpu.SemaphoreType.DMA,
            pltpu.SemaphoreType.DMA,
            pltpu.SemaphoreType.DMA,
            pltpu.SemaphoreType.DMA,
        ],
        compiler_params=pltpu.CompilerParams(use_tc_tiling_on_sc=False,
                                             needs_layout_passes=False),
    )(xt, tokP, pos_table)


def kernel(x, token_table, pos_table):
    xt = jnp.swapaxes(x, 0, 1).astype(jnp.int32)
    tokT = jnp.swapaxes(token_table, 0, 1)
    out_tdb = _embed(xt, tokT, pos_table)
    return jnp.transpose(out_tdb, (2, 0, 1))


# speedup vs baseline: 1.1913x; 1.1913x over previous
import jax
import jax.numpy as jnp
from jax import lax
from jax.experimental import pallas as pl
from jax.experimental.pallas import tpu as pltpu
from jax.experimental.pallas import tpu_sc as plsc

NC = 2
NS = 16
NW = NC * NS

VOCAB = 100000
MAXLEN = 200
D = 32
BATCH = 4096
GROUPS = BATCH // 16
NBUF = 3


def _body(xt_hbm, tokT_hbm, pos_hbm, out_hbm, row_v, idx0, idx1, idx2,
          buf0, buf1, buf2, pos_v, si0, si1, si2, ss0, ss1, ss2):
    d = lax.axis_index("s") * NC + lax.axis_index("c")

    idx = (idx0, idx1, idx2)
    bufs = (buf0, buf1, buf2)
    sem_i = (si0, si1, si2)
    sem_s = (ss0, ss1, ss2)

    pltpu.sync_copy(tokT_hbm.at[d], row_v)
    pltpu.sync_copy(pos_hbm, pos_v)

    zeros = lax.iota(jnp.int32, 16) * 0

    def step(t, r, drain, refire):
        pltpu.make_async_copy(xt_hbm.at[t], idx[r], sem_i[r]).wait()

        pb = plsc.load_gather(pos_v, [zeros + t, zeros + d])

        if drain:
            pltpu.make_async_copy(bufs[r], out_hbm.at[t - NBUF, d],
                                  sem_s[r]).wait()

        src = idx[r]
        dst = bufs[r]

        @plsc.parallel_loop(0, GROUPS, unroll=32)
        def _(g):
            iv = src[pl.ds(g * 16, 16)]
            v = plsc.load_gather(row_v, [iv])
            dst[pl.ds(g * 16, 16)] = v + pb

        pltpu.async_copy(bufs[r], out_hbm.at[t, d], sem_s[r])

        if refire:
            pltpu.async_copy(xt_hbm.at[t + NBUF], idx[r], sem_i[r])

    for r in range(NBUF):
        pltpu.async_copy(xt_hbm.at[r], idx[r], sem_i[r])

    for t in range(NBUF):
        step(t, t, drain=False, refire=True)

    def ring_body(j, _):
        t = NBUF * j
        for r in range(NBUF):
            step(t + r, r, drain=True, refire=True)
        return 0

    lax.fori_loop(1, 65, ring_body, 0)

    step(195, 0, drain=True, refire=True)
    step(196, 1, drain=True, refire=True)
    step(197, 2, drain=True, refire=False)
    step(198, 0, drain=True, refire=False)
    step(199, 1, drain=True, refire=False)

    for t in range(MAXLEN - NBUF, MAXLEN):
        pltpu.make_async_copy(bufs[t % NBUF], out_hbm.at[t, d],
                              sem_s[t % NBUF]).wait()


@jax.jit
def _embed(xt, tokT, pos_table):
    mesh = plsc.VectorSubcoreMesh(core_axis_name="c", subcore_axis_name="s")
    return pl.kernel(
        _body,
        out_type=jax.ShapeDtypeStruct((MAXLEN, D, BATCH), jnp.float32),
        mesh=mesh,
        scratch_types=[
            pltpu.VMEM((VOCAB,), jnp.float32),
            pltpu.VMEM((BATCH,), jnp.int32),
            pltpu.VMEM((BATCH,), jnp.int32),
            pltpu.VMEM((BATCH,), jnp.int32),
            pltpu.VMEM((BATCH,), jnp.float32),
            pltpu.VMEM((BATCH,), jnp.float32),
            pltpu.VMEM((BATCH,), jnp.float32),
            pltpu.VMEM((MAXLEN, D), jnp.float32),
            pltpu.SemaphoreType.DMA,
            pltpu.SemaphoreType.DMA,
            pltpu.SemaphoreType.DMA,
            pltpu.SemaphoreType.DMA,
            pltpu.SemaphoreType.DMA,
            pltpu.SemaphoreType.DMA,
        ],
        compiler_params=pltpu.CompilerParams(use_tc_tiling_on_sc=False,
                                             needs_layout_passes=False),
    )(xt, tokT, pos_table)


def kernel(x, token_table, pos_table):
    xt = jnp.swapaxes(x, 0, 1).astype(jnp.int32)
    tokT = jnp.swapaxes(token_table, 0, 1)
    out_tdb = _embed(xt, tokT, pos_table)
    return jnp.transpose(out_tdb, (2, 0, 1))
